# baseline (speedup 1.0000x reference)
import functools

import jax
import jax.numpy as jnp
from jax import lax
from jax.experimental import pallas as pl
from jax.experimental.pallas import tpu as pltpu
from jax.experimental.pallas import tpu_sc as plsc

N = 10000
E = 160000
D = 256
H = 128
NS = 16
EP = E // NS
K = 80
NCHUNK = EP // K
NP = 10240
RPT = NP // NS
RB = 2000


def _proj_body(x_ref, w_ref, asv_ref, adv_ref,
               xp0_ref, xp1_ref, as_ref, ad_ref, ws_ref):
    xp = jnp.dot(x_ref[...], w_ref[...], preferred_element_type=jnp.float32)
    xp0_ref[...] = xp[:, :H]
    xp1_ref[...] = xp[:, H:]
    a_s = jnp.sum(xp * asv_ref[...][None, :], axis=1)
    a_d = jnp.sum(xp * adv_ref[...][None, :], axis=1)
    as_ref[...] = a_s[:, None]
    ad_ref[...] = a_d[:, None]
    al = a_s + a_d
    al = jnp.where(al > 0, al, 0.2 * al)
    ws_ref[...] = jnp.exp(al)[:, None]


_proj = pl.pallas_call(
    _proj_body,
    grid=(N // RB,),
    in_specs=[
        pl.BlockSpec((RB, D), lambda i: (i, 0)),
        pl.BlockSpec((D, D), lambda i: (0, 0)),
        pl.BlockSpec((D,), lambda i: (0,)),
        pl.BlockSpec((D,), lambda i: (0,)),
    ],
    out_specs=[
        pl.BlockSpec((RB, H), lambda i: (i, 0)),
        pl.BlockSpec((RB, H), lambda i: (i, 0)),
        pl.BlockSpec((RB, 1), lambda i: (i, 0)),
        pl.BlockSpec((RB, 1), lambda i: (i, 0)),
        pl.BlockSpec((RB, 1), lambda i: (i, 0)),
    ],
    out_shape=[
        jax.ShapeDtypeStruct((N, H), jnp.float32),
        jax.ShapeDtypeStruct((N, H), jnp.float32),
        jax.ShapeDtypeStruct((N, 1), jnp.float32),
        jax.ShapeDtypeStruct((N, 1), jnp.float32),
        jax.ShapeDtypeStruct((N, 1), jnp.float32),
    ],
)


_sc_mesh = plsc.VectorSubcoreMesh(core_axis_name="c", subcore_axis_name="s")


@functools.partial(
    pl.kernel,
    out_type=[
        jax.ShapeDtypeStruct((NP, H), jnp.float32),
        jax.ShapeDtypeStruct((NP, H), jnp.float32),
        jax.ShapeDtypeStruct((NS, 1, N), jnp.float32),
    ],
    mesh=_sc_mesh,
    scratch_types=[
        pltpu.VMEM_SHARED((NP, H), jnp.float32),
        pltpu.VMEM((1, K), jnp.int32),
        pltpu.VMEM((1, K), jnp.int32),
        pltpu.VMEM((K,), jnp.float32),
        pltpu.VMEM((K,), jnp.float32),
        pltpu.VMEM((K, H), jnp.float32),
        pltpu.VMEM((1, K), jnp.int32),
        pltpu.VMEM((1, K), jnp.int32),
        pltpu.VMEM((K,), jnp.float32),
        pltpu.VMEM((K,), jnp.float32),
        pltpu.VMEM((K, H), jnp.float32),
        pltpu.VMEM((K,), jnp.float32),
        pltpu.VMEM((1, N), jnp.float32),
        pltpu.VMEM((1, K), jnp.int32),
        pltpu.VMEM((1, K), jnp.int32),
        pltpu.SemaphoreType.DMA,
        pltpu.SemaphoreType.DMA,
        pltpu.SemaphoreType.DMA,
        pltpu.SemaphoreType.DMA,
        pltpu.SemaphoreType.DMA,
        pltpu.SemaphoreType.DMA,
    ],
    compiler_params=pltpu.CompilerParams(needs_layout_passes=False),
)
def _edge_kernel(xp0_hbm, xp1_hbm, asrc_hbm, adst_hbm, src_hbm, dst_hbm,
                 num0_hbm, num1_hbm, denp_hbm,
                 acc, sidx_a, didx_a, av_a, dv_a, rows_a,
                 sidx_b, didx_b, av_b, dv_b, rows_b,
                 wbuf, den_l, dsc_a, dsc_b,
                 gsem_a, gsem_b, ssem_a, ssem_b, isem_a, isem_b):
    c = lax.axis_index("c")
    s = lax.axis_index("s")
    zero16 = jnp.zeros((16,), jnp.float32)

    def zfill(i, _):
        r = i // (H // 16)
        f = i % (H // 16)
        rows_a[r, pl.ds(f * 16, 16)] = zero16
        return 0
    lax.fori_loop(0, K * (H // 16), zfill, 0)

    def dfill(i, _):
        den_l[0, pl.ds(i * 16, 16)] = zero16
        return 0
    lax.fori_loop(0, N // 16, dfill, 0)

    row0 = s * RPT
    for j in range(RPT // K):
        pltpu.sync_copy(rows_a, acc.at[pl.ds(row0 + j * K, K)])

    def run(xp_hbm):
        base = s * NCHUNK

        def issue_idx(i, sidx, didx, isem):
            pltpu.async_copy(src_hbm.at[base + i], sidx, isem)
            pltpu.async_copy(dst_hbm.at[base + i], didx, isem)

        def wait_idx(i, sidx, didx, isem):
            pltpu.make_async_copy(src_hbm.at[base + i], sidx, isem).wait()
            pltpu.make_async_copy(dst_hbm.at[base + i], didx, isem).wait()

        def issue_gathers(sidx, didx, av, dv, rows, gsem):
            h = K // 2
            pltpu.async_copy(xp_hbm.at[sidx.at[0, pl.ds(0, h)]],
                             rows.at[pl.ds(0, h)], gsem)
            pltpu.async_copy(xp_hbm.at[sidx.at[0, pl.ds(h, h)]],
                             rows.at[pl.ds(h, h)], gsem)
            pltpu.async_copy(asrc_hbm.at[sidx.at[0]], av, gsem)
            pltpu.async_copy(adst_hbm.at[didx.at[0]], dv, gsem)

        def wait_gathers(sidx, didx, av, dv, rows, gsem):
            pltpu.make_async_copy(xp_hbm.at[sidx.at[0]], rows, gsem).wait()
            pltpu.make_async_copy(asrc_hbm.at[sidx.at[0]], av, gsem).wait()
            pltpu.make_async_copy(adst_hbm.at[didx.at[0]], dv, gsem).wait()

        def compute(didx, av, dv, rows, dsc):
            zi = jnp.zeros((16,), jnp.int32)
            for j in range(K // 16):
                di = didx[0, pl.ds(j * 16, 16)]
                dsc[0, pl.ds(j * 16, 16)] = di
                al = av[pl.ds(j * 16, 16)] + dv[pl.ds(j * 16, 16)]
                al = jnp.where(al > 0, al, 0.2 * al)
                w = jnp.exp(al)
                wbuf[pl.ds(j * 16, 16)] = w
                plsc.addupdate_scatter(den_l, [zi, di], w)

            def egrp(j, _):
                w16 = wbuf[pl.ds(j * 16, 16)]
                e0 = j * 16
                for l in range(16):
                    wv = jnp.full((16,), w16[l], jnp.float32)
                    for f in range(H // 16):
                        rows[e0 + l, pl.ds(f * 16, 16)] = (
                            rows[e0 + l, pl.ds(f * 16, 16)] * wv)
                return 0
            lax.fori_loop(0, K // 16, egrp, 0)

        issue_idx(0, sidx_a, didx_a, isem_a)
        wait_idx(0, sidx_a, didx_a, isem_a)
        issue_gathers(sidx_a, didx_a, av_a, dv_a, rows_a, gsem_a)
        issue_idx(1, sidx_b, didx_b, isem_b)

        plsc.subcore_barrier()

        def pair_body(p, _):
            c0 = 2 * p
            wait_gathers(sidx_a, didx_a, av_a, dv_a, rows_a, gsem_a)
            compute(didx_a, av_a, dv_a, rows_a, dsc_a)

            @pl.when(p > 0)
            def _():
                pltpu.make_async_copy(rows_b, acc.at[dsc_b.at[0]],
                                      ssem_b).wait()
            wait_idx(c0 + 1, sidx_b, didx_b, isem_b)
            issue_gathers(sidx_b, didx_b, av_b, dv_b, rows_b, gsem_b)
            pltpu.async_copy(rows_a, acc.at[dsc_a.at[0]], ssem_a, add=True)

            issue_idx(c0 + 2, sidx_a, didx_a, isem_a)
            wait_gathers(sidx_b, didx_b, av_b, dv_b, rows_b, gsem_b)
            compute(didx_b, av_b, dv_b, rows_b, dsc_b)

            pltpu.make_async_copy(rows_a, acc.at[dsc_a.at[0]], ssem_a).wait()
            wait_idx(c0 + 2, sidx_a, didx_a, isem_a)
            issue_gathers(sidx_a, didx_a, av_a, dv_a, rows_a, gsem_a)

            @pl.when(p + 1 < (NCHUNK - 1) // 2)
            def _():
                issue_idx(c0 + 3, sidx_b, didx_b, isem_b)
            pltpu.async_copy(rows_b, acc.at[dsc_b.at[0]], ssem_b, add=True)
            return 0
        lax.fori_loop(0, (NCHUNK - 1) // 2, pair_body, 0)

        wait_gathers(sidx_a, didx_a, av_a, dv_a, rows_a, gsem_a)
        compute(didx_a, av_a, dv_a, rows_a, dsc_a)
        pltpu.make_async_copy(rows_b, acc.at[dsc_b.at[0]], ssem_b).wait()
        pltpu.sync_copy(rows_a, acc.at[dsc_a.at[0]], add=True)

    @pl.when(c == 0)
    def _():
        run(xp0_hbm)

    @pl.when(c == 1)
    def _():
        run(xp1_hbm)

    plsc.subcore_barrier()

    @pl.when(c == 0)
    def _():
        pltpu.sync_copy(acc.at[pl.ds(row0, RPT)], num0_hbm.at[pl.ds(row0, RPT)])
        pltpu.sync_copy(den_l, denp_hbm.at[s])

    @pl.when(c == 1)
    def _():
        pltpu.sync_copy(acc.at[pl.ds(row0, RPT)], num1_hbm.at[pl.ds(row0, RPT)])


def _denred_body(dp_ref, out_ref):
    out_ref[...] = jnp.sum(dp_ref[...], axis=0)[:, None]


_denred = pl.pallas_call(
    _denred_body,
    out_shape=jax.ShapeDtypeStruct((N, 1), jnp.float32),
)


def _stats_body(n0_ref, n1_ref, x0_ref, x1_ref, dp_ref, ws_ref, b_ref,
                pre_ref, ssum_ref, ssq_ref):
    i = pl.program_id(0)
    ws = ws_ref[...][:, 0]
    den = dp_ref[...][:, 0] + ws
    num = jnp.concatenate([n0_ref[...], n1_ref[...]], axis=1)
    xp = jnp.concatenate([x0_ref[...], x1_ref[...]], axis=1)
    num = num + ws[:, None] * xp
    pre = num / (den + 1e-16)[:, None] + b_ref[...][None, :]
    pre = jnp.maximum(pre, 0.0)
    pre_ref[...] = pre
    ps = jnp.sum(pre, axis=0, keepdims=True)
    pq = jnp.sum(pre * pre, axis=0, keepdims=True)

    @pl.when(i == 0)
    def _():
        ssum_ref[...] = ps
        ssq_ref[...] = pq

    @pl.when(i > 0)
    def _():
        ssum_ref[...] += ps
        ssq_ref[...] += pq


_stats = pl.pallas_call(
    _stats_body,
    grid=(N // RB,),
    in_specs=[
        pl.BlockSpec((RB, H), lambda i: (i, 0)),
        pl.BlockSpec((RB, H), lambda i: (i, 0)),
        pl.BlockSpec((RB, H), lambda i: (i, 0)),
        pl.BlockSpec((RB, H), lambda i: (i, 0)),
        pl.BlockSpec((RB, 1), lambda i: (i, 0)),
        pl.BlockSpec((RB, 1), lambda i: (i, 0)),
        pl.BlockSpec((D,), lambda i: (0,)),
    ],
    out_specs=[
        pl.BlockSpec((RB, D), lambda i: (i, 0)),
        pl.BlockSpec((1, D), lambda i: (0, 0)),
        pl.BlockSpec((1, D), lambda i: (0, 0)),
    ],
    out_shape=[
        jax.ShapeDtypeStruct((N, D), jnp.float32),
        jax.ShapeDtypeStruct((1, D), jnp.float32),
        jax.ShapeDtypeStruct((1, D), jnp.float32),
    ],
)


def _final_body(pre_ref, x_ref, ssum_ref, ssq_ref, g_ref, b_ref, out_ref):
    mean = ssum_ref[0, :] * (1.0 / N)
    var = ssq_ref[0, :] * (1.0 / N) - mean * mean
    inv = lax.rsqrt(var + 1e-5)
    scale = inv * g_ref[...]
    out_ref[...] = ((pre_ref[...] - mean[None, :]) * scale[None, :]
                    + b_ref[...][None, :] + x_ref[...])


_final = pl.pallas_call(
    _final_body,
    grid=(N // RB,),
    in_specs=[
        pl.BlockSpec((RB, D), lambda i: (i, 0)),
        pl.BlockSpec((RB, D), lambda i: (i, 0)),
        pl.BlockSpec((1, D), lambda i: (0, 0)),
        pl.BlockSpec((1, D), lambda i: (0, 0)),
        pl.BlockSpec((D,), lambda i: (0,)),
        pl.BlockSpec((D,), lambda i: (0,)),
    ],
    out_specs=pl.BlockSpec((RB, D), lambda i: (i, 0)),
    out_shape=jax.ShapeDtypeStruct((N, D), jnp.float32),
)


def kernel(x, edge_index, W, att_src, att_dst, bias, bn_gamma, bn_beta):
    src = edge_index[0].reshape(NS * NCHUNK, 1, K)
    dst = edge_index[1].reshape(NS * NCHUNK, 1, K)
    xp0, xp1, a_s, a_d, w_self = _proj(x, W, att_src, att_dst)
    num0, num1, denp = _edge_kernel(xp0, xp1, a_s.reshape(N), a_d.reshape(N),
                                    src, dst)
    den_col = _denred(denp.reshape(NS, N))
    pre, ssum, ssq = _stats(num0, num1, xp0, xp1, den_col, w_self, bias)
    return _final(pre, x, ssum, ssq, bn_gamma, bn_beta)


# speedup vs baseline: 1.0132x; 1.0132x over previous
import functools

import jax
import jax.numpy as jnp
from jax import lax
from jax.experimental import pallas as pl
from jax.experimental.pallas import tpu as pltpu
from jax.experimental.pallas import tpu_sc as plsc

N = 10000
E = 160000
D = 256
H = 128
NS = 16
EP = E // NS
K = 80
NCHUNK = EP // K
NP = 10240
RPT = NP // NS
RB = 2000


def _proj_body(x_ref, w_ref, asv_ref, adv_ref,
               xp0_ref, xp1_ref, as_ref, ad_ref, ws_ref):
    xp = jnp.dot(x_ref[...], w_ref[...], preferred_element_type=jnp.float32)
    xp0_ref[...] = xp[:, :H]
    xp1_ref[...] = xp[:, H:]
    a_s = jnp.sum(xp * asv_ref[...][None, :], axis=1)
    a_d = jnp.sum(xp * adv_ref[...][None, :], axis=1)
    as_ref[...] = a_s[:, None]
    ad_ref[...] = a_d[:, None]
    al = a_s + a_d
    al = jnp.where(al > 0, al, 0.2 * al)
    ws_ref[...] = jnp.exp(al)[:, None]


_proj = pl.pallas_call(
    _proj_body,
    grid=(N // RB,),
    in_specs=[
        pl.BlockSpec((RB, D), lambda i: (i, 0)),
        pl.BlockSpec((D, D), lambda i: (0, 0)),
        pl.BlockSpec((D,), lambda i: (0,)),
        pl.BlockSpec((D,), lambda i: (0,)),
    ],
    out_specs=[
        pl.BlockSpec((RB, H), lambda i: (i, 0)),
        pl.BlockSpec((RB, H), lambda i: (i, 0)),
        pl.BlockSpec((RB, 1), lambda i: (i, 0)),
        pl.BlockSpec((RB, 1), lambda i: (i, 0)),
        pl.BlockSpec((RB, 1), lambda i: (i, 0)),
    ],
    out_shape=[
        jax.ShapeDtypeStruct((N, H), jnp.float32),
        jax.ShapeDtypeStruct((N, H), jnp.float32),
        jax.ShapeDtypeStruct((N, 1), jnp.float32),
        jax.ShapeDtypeStruct((N, 1), jnp.float32),
        jax.ShapeDtypeStruct((N, 1), jnp.float32),
    ],
)


_sc_mesh = plsc.VectorSubcoreMesh(core_axis_name="c", subcore_axis_name="s")


@functools.partial(
    pl.kernel,
    out_type=[
        jax.ShapeDtypeStruct((NP, H), jnp.float32),
        jax.ShapeDtypeStruct((NP, H), jnp.float32),
        jax.ShapeDtypeStruct((NS, 1, N), jnp.float32),
    ],
    mesh=_sc_mesh,
    scratch_types=[
        pltpu.VMEM_SHARED((NP, H), jnp.float32),
        pltpu.VMEM((1, K), jnp.int32),
        pltpu.VMEM((1, K), jnp.int32),
        pltpu.VMEM((K,), jnp.float32),
        pltpu.VMEM((K,), jnp.float32),
        pltpu.VMEM((K, H), jnp.float32),
        pltpu.VMEM((1, K), jnp.int32),
        pltpu.VMEM((1, K), jnp.int32),
        pltpu.VMEM((K,), jnp.float32),
        pltpu.VMEM((K,), jnp.float32),
        pltpu.VMEM((K, H), jnp.float32),
        pltpu.VMEM((K,), jnp.float32),
        pltpu.VMEM((1, N), jnp.float32),
        pltpu.VMEM((1, K), jnp.int32),
        pltpu.VMEM((1, K), jnp.int32),
        pltpu.SemaphoreType.DMA,
        pltpu.SemaphoreType.DMA,
        pltpu.SemaphoreType.DMA,
        pltpu.SemaphoreType.DMA,
        pltpu.SemaphoreType.DMA,
        pltpu.SemaphoreType.DMA,
    ],
    compiler_params=pltpu.CompilerParams(needs_layout_passes=False),
)
def _edge_kernel(xp0_hbm, xp1_hbm, asrc_hbm, adst_hbm, src_hbm, dst_hbm,
                 num0_hbm, num1_hbm, denp_hbm,
                 acc, sidx_a, didx_a, av_a, dv_a, rows_a,
                 sidx_b, didx_b, av_b, dv_b, rows_b,
                 wbuf, den_l, dsc_a, dsc_b,
                 gsem_a, gsem_b, ssem_a, ssem_b, isem_a, isem_b):
    c = lax.axis_index("c")
    s = lax.axis_index("s")
    zero16 = jnp.zeros((16,), jnp.float32)

    def zfill(i, _):
        r = i // (H // 16)
        f = i % (H // 16)
        rows_a[r, pl.ds(f * 16, 16)] = zero16
        return 0
    lax.fori_loop(0, K * (H // 16), zfill, 0)

    def dfill(i, _):
        den_l[0, pl.ds(i * 16, 16)] = zero16
        return 0
    lax.fori_loop(0, N // 16, dfill, 0)

    row0 = s * RPT
    for j in range(RPT // K):
        pltpu.sync_copy(rows_a, acc.at[pl.ds(row0 + j * K, K)])

    def run(xp_hbm):
        base = s * NCHUNK

        def issue_idx(i, sidx, didx, isem):
            pltpu.async_copy(src_hbm.at[base + i], sidx, isem)
            pltpu.async_copy(dst_hbm.at[base + i], didx, isem)

        def wait_idx(i, sidx, didx, isem):
            pltpu.make_async_copy(src_hbm.at[base + i], sidx, isem).wait()
            pltpu.make_async_copy(dst_hbm.at[base + i], didx, isem).wait()

        def issue_gathers(sidx, didx, av, dv, rows, gsem):
            pltpu.async_copy(xp_hbm.at[sidx.at[0]], rows, gsem)
            pltpu.async_copy(asrc_hbm.at[sidx.at[0]], av, gsem)
            pltpu.async_copy(adst_hbm.at[didx.at[0]], dv, gsem)

        def wait_gathers(sidx, didx, av, dv, rows, gsem):
            pltpu.make_async_copy(xp_hbm.at[sidx.at[0]], rows, gsem).wait()
            pltpu.make_async_copy(asrc_hbm.at[sidx.at[0]], av, gsem).wait()
            pltpu.make_async_copy(adst_hbm.at[didx.at[0]], dv, gsem).wait()

        def compute(didx, av, dv, rows, dsc):
            zi = jnp.zeros((16,), jnp.int32)
            for j in range(K // 16):
                di = didx[0, pl.ds(j * 16, 16)]
                dsc[0, pl.ds(j * 16, 16)] = di
                al = av[pl.ds(j * 16, 16)] + dv[pl.ds(j * 16, 16)]
                al = jnp.where(al > 0, al, 0.2 * al)
                w = jnp.exp(al)
                wbuf[pl.ds(j * 16, 16)] = w
                plsc.addupdate_scatter(den_l, [zi, di], w)

            def egrp(j, _):
                w16 = wbuf[pl.ds(j * 16, 16)]
                e0 = j * 16
                for l in range(16):
                    wv = jnp.full((16,), w16[l], jnp.float32)
                    for f in range(H // 16):
                        rows[e0 + l, pl.ds(f * 16, 16)] = (
                            rows[e0 + l, pl.ds(f * 16, 16)] * wv)
                return 0
            lax.fori_loop(0, K // 16, egrp, 0)

        issue_idx(0, sidx_a, didx_a, isem_a)
        wait_idx(0, sidx_a, didx_a, isem_a)
        issue_gathers(sidx_a, didx_a, av_a, dv_a, rows_a, gsem_a)
        issue_idx(1, sidx_b, didx_b, isem_b)

        plsc.subcore_barrier()

        def pair_body(p, _):
            c0 = 2 * p
            wait_gathers(sidx_a, didx_a, av_a, dv_a, rows_a, gsem_a)
            compute(didx_a, av_a, dv_a, rows_a, dsc_a)

            @pl.when(p > 0)
            def _():
                pltpu.make_async_copy(rows_b, acc.at[dsc_b.at[0]],
                                      ssem_b).wait()
            wait_idx(c0 + 1, sidx_b, didx_b, isem_b)
            issue_gathers(sidx_b, didx_b, av_b, dv_b, rows_b, gsem_b)
            pltpu.async_copy(rows_a, acc.at[dsc_a.at[0]], ssem_a, add=True)

            issue_idx(c0 + 2, sidx_a, didx_a, isem_a)
            wait_gathers(sidx_b, didx_b, av_b, dv_b, rows_b, gsem_b)
            compute(didx_b, av_b, dv_b, rows_b, dsc_b)

            pltpu.make_async_copy(rows_a, acc.at[dsc_a.at[0]], ssem_a).wait()
            wait_idx(c0 + 2, sidx_a, didx_a, isem_a)
            issue_gathers(sidx_a, didx_a, av_a, dv_a, rows_a, gsem_a)

            @pl.when(p + 1 < (NCHUNK - 1) // 2)
            def _():
                issue_idx(c0 + 3, sidx_b, didx_b, isem_b)
            pltpu.async_copy(rows_b, acc.at[dsc_b.at[0]], ssem_b, add=True)
            return 0
        lax.fori_loop(0, (NCHUNK - 1) // 2, pair_body, 0)

        wait_gathers(sidx_a, didx_a, av_a, dv_a, rows_a, gsem_a)
        compute(didx_a, av_a, dv_a, rows_a, dsc_a)
        pltpu.make_async_copy(rows_b, acc.at[dsc_b.at[0]], ssem_b).wait()
        pltpu.sync_copy(rows_a, acc.at[dsc_a.at[0]], add=True)

    @pl.when(c == 0)
    def _():
        run(xp0_hbm)

    @pl.when(c == 1)
    def _():
        run(xp1_hbm)

    plsc.subcore_barrier()

    @pl.when(c == 0)
    def _():
        pltpu.sync_copy(acc.at[pl.ds(row0, RPT)], num0_hbm.at[pl.ds(row0, RPT)])
        pltpu.sync_copy(den_l, denp_hbm.at[s])

    @pl.when(c == 1)
    def _():
        pltpu.sync_copy(acc.at[pl.ds(row0, RPT)], num1_hbm.at[pl.ds(row0, RPT)])


NB = N // RB


def _post_body(dp_ref, n0_ref, n1_ref, x0_ref, x1_ref, ws_ref, b_ref,
               x_ref, g_ref, bt_ref, out_ref, ssum_ref, ssq_ref, den_scr):
    g = pl.program_id(0)
    i = jnp.maximum(g - 1, 0) % NB

    @pl.when(g == 0)
    def _():
        den_scr[...] = jnp.sum(dp_ref[...], axis=0)[:, None]
        ssum_ref[...] = jnp.zeros((1, D), jnp.float32)
        ssq_ref[...] = jnp.zeros((1, D), jnp.float32)

    @pl.when(g > 0)
    def _():
        ws = ws_ref[...][:, 0]
        den = den_scr[pl.ds(i * RB, RB), :][:, 0] + ws
        num = jnp.concatenate([n0_ref[...], n1_ref[...]], axis=1)
        xp = jnp.concatenate([x0_ref[...], x1_ref[...]], axis=1)
        num = num + ws[:, None] * xp
        pre = num / (den + 1e-16)[:, None] + b_ref[...][None, :]
        pre = jnp.maximum(pre, 0.0)

        @pl.when(g <= NB)
        def _():
            ssum_ref[...] += jnp.sum(pre, axis=0, keepdims=True)
            ssq_ref[...] += jnp.sum(pre * pre, axis=0, keepdims=True)

        @pl.when(g > NB)
        def _():
            mean = ssum_ref[0, :] * (1.0 / N)
            var = ssq_ref[0, :] * (1.0 / N) - mean * mean
            scale = lax.rsqrt(var + 1e-5) * g_ref[...]
            out_ref[...] = ((pre - mean[None, :]) * scale[None, :]
                            + bt_ref[...][None, :] + x_ref[...])


_post = pl.pallas_call(
    _post_body,
    grid=(2 * NB + 1,),
    in_specs=[
        pl.BlockSpec((NS, N), lambda g: (0, 0)),
        pl.BlockSpec((RB, H), lambda g: (jnp.maximum(g - 1, 0) % NB, 0)),
        pl.BlockSpec((RB, H), lambda g: (jnp.maximum(g - 1, 0) % NB, 0)),
        pl.BlockSpec((RB, H), lambda g: (jnp.maximum(g - 1, 0) % NB, 0)),
        pl.BlockSpec((RB, H), lambda g: (jnp.maximum(g - 1, 0) % NB, 0)),
        pl.BlockSpec((RB, 1), lambda g: (jnp.maximum(g - 1, 0) % NB, 0)),
        pl.BlockSpec((D,), lambda g: (0,)),
        pl.BlockSpec((RB, D), lambda g: (jnp.maximum(g - NB - 1, 0), 0)),
        pl.BlockSpec((D,), lambda g: (0,)),
        pl.BlockSpec((D,), lambda g: (0,)),
    ],
    out_specs=[
        pl.BlockSpec((RB, D), lambda g: (jnp.maximum(g - NB - 1, 0), 0)),
        pl.BlockSpec((1, D), lambda g: (0, 0)),
        pl.BlockSpec((1, D), lambda g: (0, 0)),
    ],
    out_shape=[
        jax.ShapeDtypeStruct((N, D), jnp.float32),
        jax.ShapeDtypeStruct((1, D), jnp.float32),
        jax.ShapeDtypeStruct((1, D), jnp.float32),
    ],
    scratch_shapes=[pltpu.VMEM((N, 1), jnp.float32)],
)


def kernel(x, edge_index, W, att_src, att_dst, bias, bn_gamma, bn_beta):
    src = edge_index[0].reshape(NS * NCHUNK, 1, K)
    dst = edge_index[1].reshape(NS * NCHUNK, 1, K)
    xp0, xp1, a_s, a_d, w_self = _proj(x, W, att_src, att_dst)
    num0, num1, denp = _edge_kernel(xp0, xp1, a_s.reshape(N), a_d.reshape(N),
                                    src, dst)
    out, _, _ = _post(denp.reshape(NS, N), num0, num1, xp0, xp1,
                      w_self, bias, x, bn_gamma, bn_beta)
    return out
